# Initial kernel scaffold; baseline (speedup 1.0000x reference)
import jax
import jax.numpy as jnp
from jax.experimental import pallas as pl


def kernel(long_term_history, W, b, adj_mx, adj_u, adj_v):
    raise NotImplementedError("write your pallas kernel here")


# speedup vs baseline: 1.2394x; 1.2394x over previous
import functools

import jax
import jax.numpy as jnp
from jax.experimental import pallas as pl
from jax.experimental.pallas import tpu as pltpu

PATCH = 12
K_NEIGH = 3
EMBED = 96
LPAD = 16


def _tc_body(samp_ref, ht_ref, w_ref, b_ref, out_ref):
    n_nodes = ht_ref.shape[1]
    for n in range(n_nodes):
        rows = [ht_ref[0, n]]
        for j in range(K_NEIGH):
            s = samp_ref[n, j]
            rows.append(ht_ref[0, s])
        x = jnp.concatenate(rows, axis=0)
        acc = jax.lax.dot_general(
            w_ref[...], x,
            dimension_numbers=(((1,), (0,)), ((), ())),
            preferred_element_type=jnp.float32,
        )
        out_ref[0, n] = acc + b_ref[...]


def kernel(long_term_history, W, b, adj_mx, adj_u, adj_v):
    Bsz, N, C, T = long_term_history.shape
    P = T // PATCH
    E = W.shape[0]

    adjusted = adj_u * adj_mx + adj_v
    probs = adjusted / jnp.sum(adjusted, axis=-1, keepdims=True)
    logits = jnp.log(probs)
    gumb = jax.random.gumbel(jax.random.key(42), (K_NEIGH, N, N), jnp.float32)
    sampled = jnp.argmax(gumb + logits[None], axis=-1).T.astype(jnp.int32)

    hist = long_term_history.reshape(Bsz, N, P, PATCH)
    ht = jnp.transpose(hist, (0, 1, 3, 2))
    ht = jnp.pad(ht, ((0, 0), (0, 0), (0, LPAD - PATCH), (0, 0)))

    wp = jnp.pad(W, ((0, 0), (0, 0), (0, LPAD - PATCH)))
    wp = wp.reshape(E, (K_NEIGH + 1) * LPAD)
    b2 = b.reshape(E, 1)

    out = pl.pallas_call(
        _tc_body,
        grid=(Bsz,),
        in_specs=[
            pl.BlockSpec(memory_space=pltpu.SMEM),
            pl.BlockSpec((1, N, LPAD, P), lambda i: (i, 0, 0, 0)),
            pl.BlockSpec((E, (K_NEIGH + 1) * LPAD), lambda i: (0, 0)),
            pl.BlockSpec((E, 1), lambda i: (0, 0)),
        ],
        out_specs=pl.BlockSpec((1, N, E, P), lambda i: (i, 0, 0, 0)),
        out_shape=jax.ShapeDtypeStruct((Bsz, N, E, P), jnp.float32),
        compiler_params=pltpu.CompilerParams(
            dimension_semantics=("parallel",),
        ),
    )(sampled, ht, wp, b2)
    return out
